# Initial kernel scaffold; baseline (speedup 1.0000x reference)
import jax
import jax.numpy as jnp
from jax.experimental import pallas as pl


def kernel(x, ln1_g, ln1_b, in_proj_w, conv_w, conv_b, x_proj_w, dt_w, dt_b, A_log, Dvec, out_proj_w, act_g, act_b, gate_w, gate_b, w1, b1, w2, b2):
    raise NotImplementedError("write your pallas kernel here")


# speedup vs baseline: 10.3093x; 10.3093x over previous
import functools

import jax
import jax.numpy as jnp
from jax.experimental import pallas as pl
from jax.experimental.pallas import tpu as pltpu

B = 2
L = 768
D_MODEL = 768
D_INNER = 1536
D_STATE = 16
D_CONV = 4
DT_RANK = 48
N_EXP = 8
TOP_K = 2
D_HID = 1024

CBLK = 512
NCB = D_INNER // CBLK
TCHUNK = 8
TBLK = 512
NTB = (B * L) // TBLK


def _silu(x):
    return x * jax.nn.sigmoid(x)


def _ln1_body(x_ref, g_ref, b_ref, o_ref):
    x = x_ref[0]
    m = jnp.mean(x, axis=-1, keepdims=True)
    v = jnp.mean((x - m) ** 2, axis=-1, keepdims=True)
    o_ref[0] = (x - m) * jax.lax.rsqrt(v + 1e-5) * g_ref[0] + b_ref[0]


def _inproj_body(xn_ref, wxi_ref, wz_ref, cwt_ref, cb_ref, xc_ref, z_ref):
    xn = xn_ref[0]
    xi = jnp.dot(xn, wxi_ref[...], preferred_element_type=jnp.float32)
    z = jnp.dot(xn, wz_ref[...], preferred_element_type=jnp.float32)
    acc = jnp.broadcast_to(cb_ref[...], xi.shape)
    for k in range(D_CONV):
        sh = D_CONV - 1 - k
        if sh == 0:
            shifted = xi
        else:
            shifted = jnp.concatenate(
                [jnp.zeros((sh, xi.shape[1]), jnp.float32), xi[: L - sh]], axis=0)
        acc = acc + shifted * cwt_ref[k:k + 1, :]
    xc_ref[0] = _silu(acc)
    z_ref[0] = z


def _xproj_body(xc_ref, xpw_ref, dtw_ref, dtb_ref, dt_ref, bs_ref, cs_ref):
    xc = xc_ref[0]
    dbl = jnp.dot(xc, xpw_ref[...], preferred_element_type=jnp.float32)
    dtp = dbl[:, :DT_RANK]
    bs_ref[0] = dbl[:, DT_RANK:DT_RANK + D_STATE]
    cs_ref[0] = dbl[:, DT_RANK + D_STATE:]
    dt = jnp.dot(dtp, dtw_ref[...], preferred_element_type=jnp.float32) + dtb_ref[...]
    dt_ref[0] = jax.nn.softplus(dt)


def _scan_body(dt_ref, xc_ref, z_ref, bs_ref, cs_ref, alog_ref, dv_ref, o_ref):
    at = -jnp.exp(alog_ref[...]).T
    dv = dv_ref[...]

    def chunk(c, h):
        t0 = c * TCHUNK
        dtc = dt_ref[0, pl.ds(t0, TCHUNK), :]
        xcc = xc_ref[0, pl.ds(t0, TCHUNK), :]
        zc = z_ref[0, pl.ds(t0, TCHUNK), :]
        bct = bs_ref[0, pl.ds(t0, TCHUNK), :].T
        cct = cs_ref[0, pl.ds(t0, TCHUNK), :].T
        rows = []
        for k in range(TCHUNK):
            dtr = dtc[k:k + 1, :]
            xr = xcc[k:k + 1, :]
            bcol = bct[:, k:k + 1]
            ccol = cct[:, k:k + 1]
            h = h * jnp.exp(dtr * at) + (dtr * xr) * bcol
            rows.append(jnp.sum(h * ccol, axis=0, keepdims=True))
        yc = jnp.concatenate(rows, axis=0)
        o_ref[0, pl.ds(t0, TCHUNK), :] = (yc + xcc * dv) * _silu(zc)
        return h

    h0 = jnp.zeros((D_STATE, CBLK), jnp.float32)
    jax.lax.fori_loop(0, L // TCHUNK, chunk, h0)


def _outproj_body(y_ref, w_ref, xn_ref, o_ref):
    o_ref[0] = jnp.dot(y_ref[0], w_ref[...],
                       preferred_element_type=jnp.float32) + xn_ref[0]


def _ln2_body(x_ref, g_ref, b_ref, gw_ref, gb_ref, o_ref, comb_ref):
    x = x_ref[0]
    m = jnp.mean(x)
    v = jnp.mean((x - m) ** 2)
    t = (x - m) * jax.lax.rsqrt(v + 1e-5) * g_ref[...] + b_ref[...]
    o_ref[0] = t
    logits = jnp.dot(t, gw_ref[...], preferred_element_type=jnp.float32) + gb_ref[...]
    iota = jax.lax.broadcasted_iota(jnp.int32, logits.shape, 1)
    m0 = jnp.max(logits, axis=-1, keepdims=True)
    i0 = jnp.min(jnp.where(logits == m0, iota, N_EXP), axis=-1, keepdims=True)
    mask0 = iota == i0
    neg = jnp.where(mask0, -jnp.inf, logits)
    m1 = jnp.max(neg, axis=-1, keepdims=True)
    i1 = jnp.min(jnp.where(neg == m1, iota, N_EXP), axis=-1, keepdims=True)
    mask1 = iota == i1
    d = jnp.exp(m1 - m0)
    p0 = 1.0 / (1.0 + d)
    comb_ref[0] = jnp.where(mask0, p0, 0.0) + jnp.where(mask1, p0 * d, 0.0)


def _moe_body(tok_ref, comb_ref, out1_ref, w1_ref, b1_ref, w2_ref, b2_ref, o_ref):
    e = pl.program_id(1)
    tok = tok_ref[...]
    h = jnp.dot(tok, w1_ref[0], preferred_element_type=jnp.float32) + b1_ref[0]
    h = 0.5 * h * (1.0 + jax.lax.erf(h * 0.7071067811865476))
    eo = jnp.dot(h, w2_ref[0], preferred_element_type=jnp.float32) + b2_ref[0]
    comb = comb_ref[...]
    lane = jax.lax.broadcasted_iota(jnp.int32, comb.shape, 1)
    w = jnp.sum(jnp.where(lane == e, comb, 0.0), axis=1, keepdims=True)
    contrib = w * eo

    @pl.when(e == 0)
    def _():
        o_ref[...] = out1_ref[...] + contrib

    @pl.when(e != 0)
    def _():
        o_ref[...] += contrib


def _full(shape):
    nd = len(shape)
    return pl.BlockSpec(shape, lambda *_: (0,) * nd)


def kernel(x, ln1_g, ln1_b, in_proj_w, conv_w, conv_b, x_proj_w, dt_w, dt_b,
           A_log, Dvec, out_proj_w, act_g, act_b, gate_w, gate_b, w1, b1, w2, b2):
    f32 = jnp.float32

    xn = pl.pallas_call(
        _ln1_body,
        grid=(B,),
        in_specs=[pl.BlockSpec((1, L, D_MODEL), lambda b: (b, 0, 0)),
                  _full((1, D_MODEL)), _full((1, D_MODEL))],
        out_specs=pl.BlockSpec((1, L, D_MODEL), lambda b: (b, 0, 0)),
        out_shape=jax.ShapeDtypeStruct((B, L, D_MODEL), f32),
    )(x, ln1_g.reshape(1, -1), ln1_b.reshape(1, -1))

    wxi = in_proj_w[:, :D_INNER]
    wz = in_proj_w[:, D_INNER:]
    xc, z = pl.pallas_call(
        _inproj_body,
        grid=(B, NCB),
        in_specs=[
            pl.BlockSpec((1, L, D_MODEL), lambda b, c: (b, 0, 0)),
            pl.BlockSpec((D_MODEL, CBLK), lambda b, c: (0, c)),
            pl.BlockSpec((D_MODEL, CBLK), lambda b, c: (0, c)),
            pl.BlockSpec((D_CONV, CBLK), lambda b, c: (0, c)),
            pl.BlockSpec((1, CBLK), lambda b, c: (0, c)),
        ],
        out_specs=[pl.BlockSpec((1, L, CBLK), lambda b, c: (b, 0, c)),
                   pl.BlockSpec((1, L, CBLK), lambda b, c: (b, 0, c))],
        out_shape=[jax.ShapeDtypeStruct((B, L, D_INNER), f32),
                   jax.ShapeDtypeStruct((B, L, D_INNER), f32)],
    )(xn, wxi, wz, conv_w.T, conv_b.reshape(1, -1))

    dt, bs, cs = pl.pallas_call(
        _xproj_body,
        grid=(B,),
        in_specs=[pl.BlockSpec((1, L, D_INNER), lambda b: (b, 0, 0)),
                  _full((D_INNER, DT_RANK + 2 * D_STATE)),
                  _full((DT_RANK, D_INNER)),
                  _full((1, D_INNER))],
        out_specs=[pl.BlockSpec((1, L, D_INNER), lambda b: (b, 0, 0)),
                   pl.BlockSpec((1, L, D_STATE), lambda b: (b, 0, 0)),
                   pl.BlockSpec((1, L, D_STATE), lambda b: (b, 0, 0))],
        out_shape=[jax.ShapeDtypeStruct((B, L, D_INNER), f32),
                   jax.ShapeDtypeStruct((B, L, D_STATE), f32),
                   jax.ShapeDtypeStruct((B, L, D_STATE), f32)],
    )(xc, x_proj_w, dt_w, dt_b.reshape(1, -1))

    y = pl.pallas_call(
        _scan_body,
        grid=(B, NCB),
        in_specs=[
            pl.BlockSpec((1, L, CBLK), lambda b, c: (b, 0, c)),
            pl.BlockSpec((1, L, CBLK), lambda b, c: (b, 0, c)),
            pl.BlockSpec((1, L, CBLK), lambda b, c: (b, 0, c)),
            pl.BlockSpec((1, L, D_STATE), lambda b, c: (b, 0, 0)),
            pl.BlockSpec((1, L, D_STATE), lambda b, c: (b, 0, 0)),
            pl.BlockSpec((CBLK, D_STATE), lambda b, c: (c, 0)),
            pl.BlockSpec((1, CBLK), lambda b, c: (0, c)),
        ],
        out_specs=pl.BlockSpec((1, L, CBLK), lambda b, c: (b, 0, c)),
        out_shape=jax.ShapeDtypeStruct((B, L, D_INNER), f32),
    )(dt, xc, z, bs, cs, A_log, Dvec.reshape(1, -1))

    out1 = pl.pallas_call(
        _outproj_body,
        grid=(B,),
        in_specs=[pl.BlockSpec((1, L, D_INNER), lambda b: (b, 0, 0)),
                  _full((D_INNER, D_MODEL)),
                  pl.BlockSpec((1, L, D_MODEL), lambda b: (b, 0, 0))],
        out_specs=pl.BlockSpec((1, L, D_MODEL), lambda b: (b, 0, 0)),
        out_shape=jax.ShapeDtypeStruct((B, L, D_MODEL), f32),
    )(y, out_proj_w, xn)

    tok, comb = pl.pallas_call(
        _ln2_body,
        grid=(B,),
        in_specs=[pl.BlockSpec((1, L, D_MODEL), lambda b: (b, 0, 0)),
                  _full((L, D_MODEL)), _full((L, D_MODEL)),
                  _full((D_MODEL, N_EXP)), _full((1, N_EXP))],
        out_specs=[pl.BlockSpec((1, L, D_MODEL), lambda b: (b, 0, 0)),
                   pl.BlockSpec((1, L, N_EXP), lambda b: (b, 0, 0))],
        out_shape=[jax.ShapeDtypeStruct((B, L, D_MODEL), f32),
                   jax.ShapeDtypeStruct((B, L, N_EXP), f32)],
    )(out1, act_g.reshape(L, D_MODEL), act_b.reshape(L, D_MODEL), gate_w,
      gate_b.reshape(1, -1))

    tok2 = tok.reshape(B * L, D_MODEL)
    comb2 = comb.reshape(B * L, N_EXP)
    out1f = out1.reshape(B * L, D_MODEL)

    out = pl.pallas_call(
        _moe_body,
        grid=(NTB, N_EXP),
        in_specs=[
            pl.BlockSpec((TBLK, D_MODEL), lambda t, e: (t, 0)),
            pl.BlockSpec((TBLK, N_EXP), lambda t, e: (t, 0)),
            pl.BlockSpec((TBLK, D_MODEL), lambda t, e: (t, 0)),
            pl.BlockSpec((1, D_MODEL, D_HID), lambda t, e: (e, 0, 0)),
            pl.BlockSpec((1, 1, D_HID), lambda t, e: (e, 0, 0)),
            pl.BlockSpec((1, D_HID, D_MODEL), lambda t, e: (e, 0, 0)),
            pl.BlockSpec((1, 1, D_MODEL), lambda t, e: (e, 0, 0)),
        ],
        out_specs=pl.BlockSpec((TBLK, D_MODEL), lambda t, e: (t, 0)),
        out_shape=jax.ShapeDtypeStruct((B * L, D_MODEL), f32),
        compiler_params=pltpu.CompilerParams(
            dimension_semantics=("arbitrary", "arbitrary")),
    )(tok2, comb2, out1f, w1, b1.reshape(N_EXP, 1, D_HID), w2,
      b2.reshape(N_EXP, 1, D_MODEL))

    return out.reshape(B, L, D_MODEL)


# speedup vs baseline: 12.5069x; 1.2132x over previous
import functools

import jax
import jax.numpy as jnp
from jax.experimental import pallas as pl
from jax.experimental.pallas import tpu as pltpu

B = 2
L = 768
D_MODEL = 768
D_INNER = 1536
D_STATE = 16
D_CONV = 4
DT_RANK = 48
N_EXP = 8
TOP_K = 2
D_HID = 1024

CBLK = 512
NCB = D_INNER // CBLK
TCHUNK = 8
TBLK = 512
NTB = (B * L) // TBLK


def _silu(x):
    return x * jax.nn.sigmoid(x)


def _ln1_body(x_ref, g_ref, b_ref, o_ref):
    x = x_ref[0]
    m = jnp.mean(x, axis=-1, keepdims=True)
    v = jnp.mean((x - m) ** 2, axis=-1, keepdims=True)
    o_ref[0] = (x - m) * jax.lax.rsqrt(v + 1e-5) * g_ref[0] + b_ref[0]


def _inproj_body(xn_ref, wxi_ref, wz_ref, cwt_ref, cb_ref, xc_ref, z_ref):
    xn = xn_ref[0]
    xi = jnp.dot(xn, wxi_ref[...], preferred_element_type=jnp.float32)
    z = jnp.dot(xn, wz_ref[...], preferred_element_type=jnp.float32)
    acc = jnp.broadcast_to(cb_ref[...], xi.shape)
    for k in range(D_CONV):
        sh = D_CONV - 1 - k
        if sh == 0:
            shifted = xi
        else:
            shifted = jnp.concatenate(
                [jnp.zeros((sh, xi.shape[1]), jnp.float32), xi[: L - sh]], axis=0)
        acc = acc + shifted * cwt_ref[k:k + 1, :]
    xc_ref[0] = _silu(acc)
    z_ref[0] = z


def _xproj_body(xc_ref, xpw_ref, dtw_ref, dtb_ref, dt_ref, bs_ref, cs_ref):
    xc = xc_ref[0]
    dbl = jnp.dot(xc, xpw_ref[...], preferred_element_type=jnp.float32)
    dtp = dbl[:, :DT_RANK]
    bs_ref[0] = dbl[:, DT_RANK:DT_RANK + D_STATE]
    cs_ref[0] = dbl[:, DT_RANK + D_STATE:]
    dt = jnp.dot(dtp, dtw_ref[...], preferred_element_type=jnp.float32) + dtb_ref[...]
    dt_ref[0] = jax.nn.softplus(dt)


def _scan_body(dt_ref, xc_ref, bs_ref, cs_ref, alog_ref, dv_ref, o_ref):
    at = -jnp.exp(alog_ref[...]).T
    dv = dv_ref[...]

    def chunk(c, h):
        t0 = c * TCHUNK
        dtc = dt_ref[0, pl.ds(t0, TCHUNK), :]
        xcc = xc_ref[0, pl.ds(t0, TCHUNK), :]
        bct = bs_ref[0, pl.ds(t0, TCHUNK), :].T
        cct = cs_ref[0, pl.ds(t0, TCHUNK), :].T
        rows = []
        for k in range(TCHUNK):
            dtr = dtc[k:k + 1, :]
            xr = xcc[k:k + 1, :]
            bcol = bct[:, k:k + 1]
            ccol = cct[:, k:k + 1]
            h = h * jnp.exp(dtr * at) + (dtr * xr) * bcol
            rows.append(jnp.sum(h * ccol, axis=0, keepdims=True))
        yc = jnp.concatenate(rows, axis=0)
        o_ref[0, pl.ds(t0, TCHUNK), :] = yc + xcc * dv
        return h

    h0 = jnp.zeros((D_STATE, D_INNER), jnp.float32)
    jax.lax.fori_loop(0, L // TCHUNK, chunk, h0)


def _outproj_body(y_ref, z_ref, w_ref, xn_ref, o_ref):
    g = y_ref[0] * _silu(z_ref[0])
    o_ref[0] = jnp.dot(g, w_ref[...],
                       preferred_element_type=jnp.float32) + xn_ref[0]


def _ln2_body(x_ref, g_ref, b_ref, gw_ref, gb_ref, o_ref, comb_ref):
    x = x_ref[0]
    m = jnp.mean(x)
    v = jnp.mean((x - m) ** 2)
    t = (x - m) * jax.lax.rsqrt(v + 1e-5) * g_ref[...] + b_ref[...]
    o_ref[0] = t
    logits = jnp.dot(t, gw_ref[...], preferred_element_type=jnp.float32) + gb_ref[...]
    iota = jax.lax.broadcasted_iota(jnp.int32, logits.shape, 1)
    m0 = jnp.max(logits, axis=-1, keepdims=True)
    i0 = jnp.min(jnp.where(logits == m0, iota, N_EXP), axis=-1, keepdims=True)
    mask0 = iota == i0
    neg = jnp.where(mask0, -jnp.inf, logits)
    m1 = jnp.max(neg, axis=-1, keepdims=True)
    i1 = jnp.min(jnp.where(neg == m1, iota, N_EXP), axis=-1, keepdims=True)
    mask1 = iota == i1
    d = jnp.exp(m1 - m0)
    p0 = 1.0 / (1.0 + d)
    comb_ref[0] = jnp.where(mask0, p0, 0.0) + jnp.where(mask1, p0 * d, 0.0)


def _moe_body(tok_ref, comb_ref, out1_ref, w1_ref, b1_ref, w2_ref, b2_ref, o_ref):
    e = pl.program_id(1)
    tok = tok_ref[...]
    h = jnp.dot(tok, w1_ref[0], preferred_element_type=jnp.float32) + b1_ref[0]
    h = 0.5 * h * (1.0 + jax.lax.erf(h * 0.7071067811865476))
    eo = jnp.dot(h, w2_ref[0], preferred_element_type=jnp.float32) + b2_ref[0]
    comb = comb_ref[...]
    lane = jax.lax.broadcasted_iota(jnp.int32, comb.shape, 1)
    w = jnp.sum(jnp.where(lane == e, comb, 0.0), axis=1, keepdims=True)
    contrib = w * eo

    @pl.when(e == 0)
    def _():
        o_ref[...] = out1_ref[...] + contrib

    @pl.when(e != 0)
    def _():
        o_ref[...] += contrib


def _full(shape):
    nd = len(shape)
    return pl.BlockSpec(shape, lambda *_: (0,) * nd)


def kernel(x, ln1_g, ln1_b, in_proj_w, conv_w, conv_b, x_proj_w, dt_w, dt_b,
           A_log, Dvec, out_proj_w, act_g, act_b, gate_w, gate_b, w1, b1, w2, b2):
    f32 = jnp.float32

    xn = pl.pallas_call(
        _ln1_body,
        grid=(B,),
        in_specs=[pl.BlockSpec((1, L, D_MODEL), lambda b: (b, 0, 0)),
                  _full((1, D_MODEL)), _full((1, D_MODEL))],
        out_specs=pl.BlockSpec((1, L, D_MODEL), lambda b: (b, 0, 0)),
        out_shape=jax.ShapeDtypeStruct((B, L, D_MODEL), f32),
    )(x, ln1_g.reshape(1, -1), ln1_b.reshape(1, -1))

    wxi = in_proj_w[:, :D_INNER]
    wz = in_proj_w[:, D_INNER:]
    xc, z = pl.pallas_call(
        _inproj_body,
        grid=(B, NCB),
        in_specs=[
            pl.BlockSpec((1, L, D_MODEL), lambda b, c: (b, 0, 0)),
            pl.BlockSpec((D_MODEL, CBLK), lambda b, c: (0, c)),
            pl.BlockSpec((D_MODEL, CBLK), lambda b, c: (0, c)),
            pl.BlockSpec((D_CONV, CBLK), lambda b, c: (0, c)),
            pl.BlockSpec((1, CBLK), lambda b, c: (0, c)),
        ],
        out_specs=[pl.BlockSpec((1, L, CBLK), lambda b, c: (b, 0, c)),
                   pl.BlockSpec((1, L, CBLK), lambda b, c: (b, 0, c))],
        out_shape=[jax.ShapeDtypeStruct((B, L, D_INNER), f32),
                   jax.ShapeDtypeStruct((B, L, D_INNER), f32)],
    )(xn, wxi, wz, conv_w.T, conv_b.reshape(1, -1))

    dt, bs, cs = pl.pallas_call(
        _xproj_body,
        grid=(B,),
        in_specs=[pl.BlockSpec((1, L, D_INNER), lambda b: (b, 0, 0)),
                  _full((D_INNER, DT_RANK + 2 * D_STATE)),
                  _full((DT_RANK, D_INNER)),
                  _full((1, D_INNER))],
        out_specs=[pl.BlockSpec((1, L, D_INNER), lambda b: (b, 0, 0)),
                   pl.BlockSpec((1, L, D_STATE), lambda b: (b, 0, 0)),
                   pl.BlockSpec((1, L, D_STATE), lambda b: (b, 0, 0))],
        out_shape=[jax.ShapeDtypeStruct((B, L, D_INNER), f32),
                   jax.ShapeDtypeStruct((B, L, D_STATE), f32),
                   jax.ShapeDtypeStruct((B, L, D_STATE), f32)],
    )(xc, x_proj_w, dt_w, dt_b.reshape(1, -1))

    y = pl.pallas_call(
        _scan_body,
        grid=(B,),
        in_specs=[
            pl.BlockSpec((1, L, D_INNER), lambda b: (b, 0, 0)),
            pl.BlockSpec((1, L, D_INNER), lambda b: (b, 0, 0)),
            pl.BlockSpec((1, L, D_STATE), lambda b: (b, 0, 0)),
            pl.BlockSpec((1, L, D_STATE), lambda b: (b, 0, 0)),
            _full((D_INNER, D_STATE)),
            _full((1, D_INNER)),
        ],
        out_specs=pl.BlockSpec((1, L, D_INNER), lambda b: (b, 0, 0)),
        out_shape=jax.ShapeDtypeStruct((B, L, D_INNER), f32),
    )(dt, xc, bs, cs, A_log, Dvec.reshape(1, -1))

    out1 = pl.pallas_call(
        _outproj_body,
        grid=(B,),
        in_specs=[pl.BlockSpec((1, L, D_INNER), lambda b: (b, 0, 0)),
                  pl.BlockSpec((1, L, D_INNER), lambda b: (b, 0, 0)),
                  _full((D_INNER, D_MODEL)),
                  pl.BlockSpec((1, L, D_MODEL), lambda b: (b, 0, 0))],
        out_specs=pl.BlockSpec((1, L, D_MODEL), lambda b: (b, 0, 0)),
        out_shape=jax.ShapeDtypeStruct((B, L, D_MODEL), f32),
    )(y, z, out_proj_w, xn)

    tok, comb = pl.pallas_call(
        _ln2_body,
        grid=(B,),
        in_specs=[pl.BlockSpec((1, L, D_MODEL), lambda b: (b, 0, 0)),
                  _full((L, D_MODEL)), _full((L, D_MODEL)),
                  _full((D_MODEL, N_EXP)), _full((1, N_EXP))],
        out_specs=[pl.BlockSpec((1, L, D_MODEL), lambda b: (b, 0, 0)),
                   pl.BlockSpec((1, L, N_EXP), lambda b: (b, 0, 0))],
        out_shape=[jax.ShapeDtypeStruct((B, L, D_MODEL), f32),
                   jax.ShapeDtypeStruct((B, L, N_EXP), f32)],
    )(out1, act_g.reshape(L, D_MODEL), act_b.reshape(L, D_MODEL), gate_w,
      gate_b.reshape(1, -1))

    tok2 = tok.reshape(B * L, D_MODEL)
    comb2 = comb.reshape(B * L, N_EXP)
    out1f = out1.reshape(B * L, D_MODEL)

    out = pl.pallas_call(
        _moe_body,
        grid=(NTB, N_EXP),
        in_specs=[
            pl.BlockSpec((TBLK, D_MODEL), lambda t, e: (t, 0)),
            pl.BlockSpec((TBLK, N_EXP), lambda t, e: (t, 0)),
            pl.BlockSpec((TBLK, D_MODEL), lambda t, e: (t, 0)),
            pl.BlockSpec((1, D_MODEL, D_HID), lambda t, e: (e, 0, 0)),
            pl.BlockSpec((1, 1, D_HID), lambda t, e: (e, 0, 0)),
            pl.BlockSpec((1, D_HID, D_MODEL), lambda t, e: (e, 0, 0)),
            pl.BlockSpec((1, 1, D_MODEL), lambda t, e: (e, 0, 0)),
        ],
        out_specs=pl.BlockSpec((TBLK, D_MODEL), lambda t, e: (t, 0)),
        out_shape=jax.ShapeDtypeStruct((B * L, D_MODEL), f32),
        compiler_params=pltpu.CompilerParams(
            dimension_semantics=("arbitrary", "arbitrary")),
    )(tok2, comb2, out1f, w1, b1.reshape(N_EXP, 1, D_HID), w2,
      b2.reshape(N_EXP, 1, D_MODEL))

    return out.reshape(B, L, D_MODEL)


# speedup vs baseline: 13.3900x; 1.0706x over previous
import functools

import jax
import jax.numpy as jnp
from jax.experimental import pallas as pl
from jax.experimental.pallas import tpu as pltpu

B = 2
L = 768
D_MODEL = 768
D_INNER = 1536
D_STATE = 16
D_CONV = 4
DT_RANK = 48
N_EXP = 8
TOP_K = 2
D_HID = 1024

CBLK = 512
NCB = D_INNER // CBLK
TCHUNK = 8
TBLK = 512
NTB = (B * L) // TBLK


def _silu(x):
    return x * jax.nn.sigmoid(x)


def _ln1_body(x_ref, g_ref, b_ref, o_ref):
    x = x_ref[0]
    m = jnp.mean(x, axis=-1, keepdims=True)
    v = jnp.mean((x - m) ** 2, axis=-1, keepdims=True)
    o_ref[0] = (x - m) * jax.lax.rsqrt(v + 1e-5) * g_ref[0] + b_ref[0]


def _inproj_body(xn_ref, wxi_ref, wz_ref, cwt_ref, cb_ref, xc_ref, z_ref):
    xn = xn_ref[0]
    xi = jnp.dot(xn, wxi_ref[...], preferred_element_type=jnp.float32)
    z = jnp.dot(xn, wz_ref[...], preferred_element_type=jnp.float32)
    acc = jnp.broadcast_to(cb_ref[...], xi.shape)
    for k in range(D_CONV):
        sh = D_CONV - 1 - k
        if sh == 0:
            shifted = xi
        else:
            shifted = jnp.concatenate(
                [jnp.zeros((sh, xi.shape[1]), jnp.float32), xi[: L - sh]], axis=0)
        acc = acc + shifted * cwt_ref[k:k + 1, :]
    xc_ref[0] = _silu(acc)
    z_ref[0] = z


def _mid_body(xc_ref, z_ref, xn_ref, xpw_ref, dtw_ref, dtb_ref, alog_ref,
              dv_ref, opw_ref, g2_ref, b2_ref, gw_ref, gb_ref,
              out1_ref, tok_ref, comb_ref, at_ref, dt_ref, bs_ref, cs_ref,
              y_ref):
    xc = xc_ref[0]
    dbl = jnp.dot(xc, xpw_ref[...], preferred_element_type=jnp.float32)
    bs_ref[...] = dbl[:, DT_RANK:DT_RANK + D_STATE]
    cs_ref[...] = dbl[:, DT_RANK + D_STATE:]
    dtm = jnp.dot(dbl[:, :DT_RANK], dtw_ref[...],
                  preferred_element_type=jnp.float32) + dtb_ref[...]
    dt_ref[...] = jax.nn.softplus(dtm)
    at_ref[...] = -jnp.exp(alog_ref[...]).T

    def chunk(c, hs):
        t0 = c * TCHUNK
        bct = bs_ref[pl.ds(t0, TCHUNK), :].T
        cct = cs_ref[pl.ds(t0, TCHUNK), :].T
        out = []
        for s in range(NCB):
            cb = slice(s * CBLK, (s + 1) * CBLK)
            h = hs[s]
            at = at_ref[:, cb]
            dv = dv_ref[:, cb]
            dtc = dt_ref[pl.ds(t0, TCHUNK), cb]
            xcc = xc_ref[0, pl.ds(t0, TCHUNK), cb]
            rows = []
            for k in range(TCHUNK):
                dtr = dtc[k:k + 1, :]
                xr = xcc[k:k + 1, :]
                bcol = bct[:, k:k + 1]
                ccol = cct[:, k:k + 1]
                h = h * jnp.exp(dtr * at) + (dtr * xr) * bcol
                rows.append(jnp.sum(h * ccol, axis=0, keepdims=True))
            yc = jnp.concatenate(rows, axis=0)
            y_ref[pl.ds(t0, TCHUNK), cb] = yc + xcc * dv
            out.append(h)
        return tuple(out)

    h0 = tuple(jnp.zeros((D_STATE, CBLK), jnp.float32) for _ in range(NCB))
    jax.lax.fori_loop(0, L // TCHUNK, chunk, h0)

    g = y_ref[...] * _silu(z_ref[0])
    out1 = jnp.dot(g, opw_ref[...],
                   preferred_element_type=jnp.float32) + xn_ref[0]
    out1_ref[0] = out1

    m = jnp.mean(out1)
    v = jnp.mean((out1 - m) ** 2)
    t = (out1 - m) * jax.lax.rsqrt(v + 1e-5) * g2_ref[...] + b2_ref[...]
    tok_ref[0] = t
    logits = jnp.dot(t, gw_ref[...], preferred_element_type=jnp.float32) + gb_ref[...]
    iota = jax.lax.broadcasted_iota(jnp.int32, logits.shape, 1)
    m0 = jnp.max(logits, axis=-1, keepdims=True)
    i0 = jnp.min(jnp.where(logits == m0, iota, N_EXP), axis=-1, keepdims=True)
    mask0 = iota == i0
    neg = jnp.where(mask0, -jnp.inf, logits)
    m1 = jnp.max(neg, axis=-1, keepdims=True)
    i1 = jnp.min(jnp.where(neg == m1, iota, N_EXP), axis=-1, keepdims=True)
    mask1 = iota == i1
    d = jnp.exp(m1 - m0)
    p0 = 1.0 / (1.0 + d)
    comb_ref[0] = jnp.where(mask0, p0, 0.0) + jnp.where(mask1, p0 * d, 0.0)


def _moe_body(tok_ref, comb_ref, out1_ref, w1_ref, b1_ref, w2_ref, b2_ref, o_ref):
    e = pl.program_id(1)
    tok = tok_ref[...]
    h = jnp.dot(tok, w1_ref[0], preferred_element_type=jnp.float32) + b1_ref[0]
    h = 0.5 * h * (1.0 + jax.lax.erf(h * 0.7071067811865476))
    eo = jnp.dot(h, w2_ref[0], preferred_element_type=jnp.float32) + b2_ref[0]
    comb = comb_ref[...]
    lane = jax.lax.broadcasted_iota(jnp.int32, comb.shape, 1)
    w = jnp.sum(jnp.where(lane == e, comb, 0.0), axis=1, keepdims=True)
    contrib = w * eo

    @pl.when(e == 0)
    def _():
        o_ref[...] = out1_ref[...] + contrib

    @pl.when(e != 0)
    def _():
        o_ref[...] += contrib


def _full(shape):
    nd = len(shape)
    return pl.BlockSpec(shape, lambda *_: (0,) * nd)


def kernel(x, ln1_g, ln1_b, in_proj_w, conv_w, conv_b, x_proj_w, dt_w, dt_b,
           A_log, Dvec, out_proj_w, act_g, act_b, gate_w, gate_b, w1, b1, w2, b2):
    f32 = jnp.float32

    xn = pl.pallas_call(
        _ln1_body,
        grid=(B,),
        in_specs=[pl.BlockSpec((1, L, D_MODEL), lambda b: (b, 0, 0)),
                  _full((1, D_MODEL)), _full((1, D_MODEL))],
        out_specs=pl.BlockSpec((1, L, D_MODEL), lambda b: (b, 0, 0)),
        out_shape=jax.ShapeDtypeStruct((B, L, D_MODEL), f32),
    )(x, ln1_g.reshape(1, -1), ln1_b.reshape(1, -1))

    wxi = in_proj_w[:, :D_INNER]
    wz = in_proj_w[:, D_INNER:]
    xc, z = pl.pallas_call(
        _inproj_body,
        grid=(B, NCB),
        in_specs=[
            pl.BlockSpec((1, L, D_MODEL), lambda b, c: (b, 0, 0)),
            pl.BlockSpec((D_MODEL, CBLK), lambda b, c: (0, c)),
            pl.BlockSpec((D_MODEL, CBLK), lambda b, c: (0, c)),
            pl.BlockSpec((D_CONV, CBLK), lambda b, c: (0, c)),
            pl.BlockSpec((1, CBLK), lambda b, c: (0, c)),
        ],
        out_specs=[pl.BlockSpec((1, L, CBLK), lambda b, c: (b, 0, c)),
                   pl.BlockSpec((1, L, CBLK), lambda b, c: (b, 0, c))],
        out_shape=[jax.ShapeDtypeStruct((B, L, D_INNER), f32),
                   jax.ShapeDtypeStruct((B, L, D_INNER), f32)],
    )(xn, wxi, wz, conv_w.T, conv_b.reshape(1, -1))

    out1, tok, comb = pl.pallas_call(
        _mid_body,
        grid=(B,),
        in_specs=[
            pl.BlockSpec((1, L, D_INNER), lambda b: (b, 0, 0)),
            pl.BlockSpec((1, L, D_INNER), lambda b: (b, 0, 0)),
            pl.BlockSpec((1, L, D_MODEL), lambda b: (b, 0, 0)),
            _full((D_INNER, DT_RANK + 2 * D_STATE)),
            _full((DT_RANK, D_INNER)),
            _full((1, D_INNER)),
            _full((D_INNER, D_STATE)),
            _full((1, D_INNER)),
            _full((D_INNER, D_MODEL)),
            _full((L, D_MODEL)), _full((L, D_MODEL)),
            _full((D_MODEL, N_EXP)), _full((1, N_EXP)),
        ],
        out_specs=[pl.BlockSpec((1, L, D_MODEL), lambda b: (b, 0, 0)),
                   pl.BlockSpec((1, L, D_MODEL), lambda b: (b, 0, 0)),
                   pl.BlockSpec((1, L, N_EXP), lambda b: (b, 0, 0))],
        out_shape=[jax.ShapeDtypeStruct((B, L, D_MODEL), f32),
                   jax.ShapeDtypeStruct((B, L, D_MODEL), f32),
                   jax.ShapeDtypeStruct((B, L, N_EXP), f32)],
        scratch_shapes=[pltpu.VMEM((D_STATE, D_INNER), f32),
                        pltpu.VMEM((L, D_INNER), f32),
                        pltpu.VMEM((L, D_STATE), f32),
                        pltpu.VMEM((L, D_STATE), f32),
                        pltpu.VMEM((L, D_INNER), f32)],
    )(xc, z, xn, x_proj_w, dt_w, dt_b.reshape(1, -1), A_log,
      Dvec.reshape(1, -1), out_proj_w, act_g.reshape(L, D_MODEL),
      act_b.reshape(L, D_MODEL), gate_w, gate_b.reshape(1, -1))

    tok2 = tok.reshape(B * L, D_MODEL)
    comb2 = comb.reshape(B * L, N_EXP)
    out1f = out1.reshape(B * L, D_MODEL)

    out = pl.pallas_call(
        _moe_body,
        grid=(NTB, N_EXP),
        in_specs=[
            pl.BlockSpec((TBLK, D_MODEL), lambda t, e: (t, 0)),
            pl.BlockSpec((TBLK, N_EXP), lambda t, e: (t, 0)),
            pl.BlockSpec((TBLK, D_MODEL), lambda t, e: (t, 0)),
            pl.BlockSpec((1, D_MODEL, D_HID), lambda t, e: (e, 0, 0)),
            pl.BlockSpec((1, 1, D_HID), lambda t, e: (e, 0, 0)),
            pl.BlockSpec((1, D_HID, D_MODEL), lambda t, e: (e, 0, 0)),
            pl.BlockSpec((1, 1, D_MODEL), lambda t, e: (e, 0, 0)),
        ],
        out_specs=pl.BlockSpec((TBLK, D_MODEL), lambda t, e: (t, 0)),
        out_shape=jax.ShapeDtypeStruct((B * L, D_MODEL), f32),
        compiler_params=pltpu.CompilerParams(
            dimension_semantics=("arbitrary", "arbitrary")),
    )(tok2, comb2, out1f, w1, b1.reshape(N_EXP, 1, D_HID), w2,
      b2.reshape(N_EXP, 1, D_MODEL))

    return out.reshape(B, L, D_MODEL)


# speedup vs baseline: 14.0246x; 1.0474x over previous
import functools

import jax
import jax.numpy as jnp
from jax.experimental import pallas as pl
from jax.experimental.pallas import tpu as pltpu

B = 2
L = 768
D_MODEL = 768
D_INNER = 1536
D_STATE = 16
D_CONV = 4
DT_RANK = 48
N_EXP = 8
TOP_K = 2
D_HID = 1024

CBLK = 512
NCB = D_INNER // CBLK
TCHUNK = 8
TBLK = 1536
NTB = (B * L) // TBLK


def _silu(x):
    return x * jax.nn.sigmoid(x)


def _ln1_body(x_ref, g_ref, b_ref, o_ref):
    x = x_ref[0]
    m = jnp.mean(x, axis=-1, keepdims=True)
    v = jnp.mean((x - m) ** 2, axis=-1, keepdims=True)
    o_ref[0] = (x - m) * jax.lax.rsqrt(v + 1e-5) * g_ref[0] + b_ref[0]


def _inproj_body(xn_ref, wxi_ref, wz_ref, cwt_ref, cb_ref, xc_ref, z_ref):
    xn = xn_ref[0]
    xi = jnp.dot(xn, wxi_ref[...], preferred_element_type=jnp.float32)
    z = jnp.dot(xn, wz_ref[...], preferred_element_type=jnp.float32)
    acc = jnp.broadcast_to(cb_ref[...], xi.shape)
    for k in range(D_CONV):
        sh = D_CONV - 1 - k
        if sh == 0:
            shifted = xi
        else:
            shifted = jnp.concatenate(
                [jnp.zeros((sh, xi.shape[1]), jnp.float32), xi[: L - sh]], axis=0)
        acc = acc + shifted * cwt_ref[k:k + 1, :]
    xc_ref[0] = _silu(acc)
    z_ref[0] = z


def _mid_body(xc_ref, z_ref, xn_ref, xpw_ref, dtw_ref, dtb_ref, alog_ref,
              dv_ref, opw_ref, g2_ref, b2_ref, gw_ref, gb_ref,
              out1_ref, tok_ref, comb_ref, at_ref, dt_ref, bs_ref, cs_ref,
              y_ref):
    xc = xc_ref[0]
    dbl = jnp.dot(xc, xpw_ref[...], preferred_element_type=jnp.float32)
    bs_ref[...] = dbl[:, DT_RANK:DT_RANK + D_STATE]
    cs_ref[...] = dbl[:, DT_RANK + D_STATE:]
    dtm = jnp.dot(dbl[:, :DT_RANK], dtw_ref[...],
                  preferred_element_type=jnp.float32) + dtb_ref[...]
    dt_ref[...] = jax.nn.softplus(dtm)
    at_ref[...] = -jnp.exp(alog_ref[...]).T

    def chunk(c, hs):
        t0 = c * TCHUNK
        bct = bs_ref[pl.ds(t0, TCHUNK), :].T
        cct = cs_ref[pl.ds(t0, TCHUNK), :].T
        out = []
        for s in range(NCB):
            cb = slice(s * CBLK, (s + 1) * CBLK)
            h = hs[s]
            at = at_ref[:, cb]
            dv = dv_ref[:, cb]
            dtc = dt_ref[pl.ds(t0, TCHUNK), cb]
            xcc = xc_ref[0, pl.ds(t0, TCHUNK), cb]
            rows = []
            for k in range(TCHUNK):
                dtr = dtc[k:k + 1, :]
                xr = xcc[k:k + 1, :]
                bcol = bct[:, k:k + 1]
                ccol = cct[:, k:k + 1]
                h = h * jnp.exp(dtr * at) + (dtr * xr) * bcol
                rows.append(jnp.sum(h * ccol, axis=0, keepdims=True))
            yc = jnp.concatenate(rows, axis=0)
            y_ref[pl.ds(t0, TCHUNK), cb] = yc + xcc * dv
            out.append(h)
        return tuple(out)

    h0 = tuple(jnp.zeros((D_STATE, CBLK), jnp.float32) for _ in range(NCB))
    jax.lax.fori_loop(0, L // TCHUNK, chunk, h0)

    g = y_ref[...] * _silu(z_ref[0])
    out1 = jnp.dot(g, opw_ref[...],
                   preferred_element_type=jnp.float32) + xn_ref[0]
    out1_ref[0] = out1

    m = jnp.mean(out1)
    v = jnp.mean((out1 - m) ** 2)
    t = (out1 - m) * jax.lax.rsqrt(v + 1e-5) * g2_ref[...] + b2_ref[...]
    tok_ref[0] = t
    logits = jnp.dot(t, gw_ref[...], preferred_element_type=jnp.float32) + gb_ref[...]
    iota = jax.lax.broadcasted_iota(jnp.int32, logits.shape, 1)
    m0 = jnp.max(logits, axis=-1, keepdims=True)
    i0 = jnp.min(jnp.where(logits == m0, iota, N_EXP), axis=-1, keepdims=True)
    mask0 = iota == i0
    neg = jnp.where(mask0, -jnp.inf, logits)
    m1 = jnp.max(neg, axis=-1, keepdims=True)
    i1 = jnp.min(jnp.where(neg == m1, iota, N_EXP), axis=-1, keepdims=True)
    mask1 = iota == i1
    d = jnp.exp(m1 - m0)
    p0 = 1.0 / (1.0 + d)
    comb_ref[0] = jnp.where(mask0, p0, 0.0) + jnp.where(mask1, p0 * d, 0.0)


def _moe_body(tok_ref, comb_ref, out1_ref, w1_ref, b1_ref, w2_ref, b2_ref, o_ref):
    e = pl.program_id(1)
    tok = tok_ref[...]
    h = jnp.dot(tok, w1_ref[0], preferred_element_type=jnp.float32) + b1_ref[0]
    h = 0.5 * h * (1.0 + jax.lax.erf(h * 0.7071067811865476))
    eo = jnp.dot(h, w2_ref[0], preferred_element_type=jnp.float32) + b2_ref[0]
    comb = comb_ref[...]
    lane = jax.lax.broadcasted_iota(jnp.int32, comb.shape, 1)
    w = jnp.sum(jnp.where(lane == e, comb, 0.0), axis=1, keepdims=True)
    contrib = w * eo

    @pl.when(e == 0)
    def _():
        o_ref[...] = out1_ref[...] + contrib

    @pl.when(e != 0)
    def _():
        o_ref[...] += contrib


def _full(shape):
    nd = len(shape)
    return pl.BlockSpec(shape, lambda *_: (0,) * nd)


def kernel(x, ln1_g, ln1_b, in_proj_w, conv_w, conv_b, x_proj_w, dt_w, dt_b,
           A_log, Dvec, out_proj_w, act_g, act_b, gate_w, gate_b, w1, b1, w2, b2):
    f32 = jnp.float32

    xn = pl.pallas_call(
        _ln1_body,
        grid=(B,),
        in_specs=[pl.BlockSpec((1, L, D_MODEL), lambda b: (b, 0, 0)),
                  _full((1, D_MODEL)), _full((1, D_MODEL))],
        out_specs=pl.BlockSpec((1, L, D_MODEL), lambda b: (b, 0, 0)),
        out_shape=jax.ShapeDtypeStruct((B, L, D_MODEL), f32),
    )(x, ln1_g.reshape(1, -1), ln1_b.reshape(1, -1))

    wxi = in_proj_w[:, :D_INNER]
    wz = in_proj_w[:, D_INNER:]
    xc, z = pl.pallas_call(
        _inproj_body,
        grid=(B, NCB),
        in_specs=[
            pl.BlockSpec((1, L, D_MODEL), lambda b, c: (b, 0, 0)),
            pl.BlockSpec((D_MODEL, CBLK), lambda b, c: (0, c)),
            pl.BlockSpec((D_MODEL, CBLK), lambda b, c: (0, c)),
            pl.BlockSpec((D_CONV, CBLK), lambda b, c: (0, c)),
            pl.BlockSpec((1, CBLK), lambda b, c: (0, c)),
        ],
        out_specs=[pl.BlockSpec((1, L, CBLK), lambda b, c: (b, 0, c)),
                   pl.BlockSpec((1, L, CBLK), lambda b, c: (b, 0, c))],
        out_shape=[jax.ShapeDtypeStruct((B, L, D_INNER), f32),
                   jax.ShapeDtypeStruct((B, L, D_INNER), f32)],
    )(xn, wxi, wz, conv_w.T, conv_b.reshape(1, -1))

    out1, tok, comb = pl.pallas_call(
        _mid_body,
        grid=(B,),
        in_specs=[
            pl.BlockSpec((1, L, D_INNER), lambda b: (b, 0, 0)),
            pl.BlockSpec((1, L, D_INNER), lambda b: (b, 0, 0)),
            pl.BlockSpec((1, L, D_MODEL), lambda b: (b, 0, 0)),
            _full((D_INNER, DT_RANK + 2 * D_STATE)),
            _full((DT_RANK, D_INNER)),
            _full((1, D_INNER)),
            _full((D_INNER, D_STATE)),
            _full((1, D_INNER)),
            _full((D_INNER, D_MODEL)),
            _full((L, D_MODEL)), _full((L, D_MODEL)),
            _full((D_MODEL, N_EXP)), _full((1, N_EXP)),
        ],
        out_specs=[pl.BlockSpec((1, L, D_MODEL), lambda b: (b, 0, 0)),
                   pl.BlockSpec((1, L, D_MODEL), lambda b: (b, 0, 0)),
                   pl.BlockSpec((1, L, N_EXP), lambda b: (b, 0, 0))],
        out_shape=[jax.ShapeDtypeStruct((B, L, D_MODEL), f32),
                   jax.ShapeDtypeStruct((B, L, D_MODEL), f32),
                   jax.ShapeDtypeStruct((B, L, N_EXP), f32)],
        scratch_shapes=[pltpu.VMEM((D_STATE, D_INNER), f32),
                        pltpu.VMEM((L, D_INNER), f32),
                        pltpu.VMEM((L, D_STATE), f32),
                        pltpu.VMEM((L, D_STATE), f32),
                        pltpu.VMEM((L, D_INNER), f32)],
    )(xc, z, xn, x_proj_w, dt_w, dt_b.reshape(1, -1), A_log,
      Dvec.reshape(1, -1), out_proj_w, act_g.reshape(L, D_MODEL),
      act_b.reshape(L, D_MODEL), gate_w, gate_b.reshape(1, -1))

    tok2 = tok.reshape(B * L, D_MODEL)
    comb2 = comb.reshape(B * L, N_EXP)
    out1f = out1.reshape(B * L, D_MODEL)

    out = pl.pallas_call(
        _moe_body,
        grid=(NTB, N_EXP),
        in_specs=[
            pl.BlockSpec((TBLK, D_MODEL), lambda t, e: (t, 0)),
            pl.BlockSpec((TBLK, N_EXP), lambda t, e: (t, 0)),
            pl.BlockSpec((TBLK, D_MODEL), lambda t, e: (t, 0)),
            pl.BlockSpec((1, D_MODEL, D_HID), lambda t, e: (e, 0, 0)),
            pl.BlockSpec((1, 1, D_HID), lambda t, e: (e, 0, 0)),
            pl.BlockSpec((1, D_HID, D_MODEL), lambda t, e: (e, 0, 0)),
            pl.BlockSpec((1, 1, D_MODEL), lambda t, e: (e, 0, 0)),
        ],
        out_specs=pl.BlockSpec((TBLK, D_MODEL), lambda t, e: (t, 0)),
        out_shape=jax.ShapeDtypeStruct((B * L, D_MODEL), f32),
        compiler_params=pltpu.CompilerParams(
            dimension_semantics=("arbitrary", "arbitrary")),
    )(tok2, comb2, out1f, w1, b1.reshape(N_EXP, 1, D_HID), w2,
      b2.reshape(N_EXP, 1, D_MODEL))

    return out.reshape(B, L, D_MODEL)
